# baseline (speedup 1.0000x reference)
import functools

import jax
import jax.numpy as jnp
from jax import lax
from jax.experimental import pallas as pl
from jax.experimental.pallas import tpu as pltpu
from jax.experimental.pallas import tpu_sc as plsc

TRUNC_R = 0.85
NEG_FILL = -70.0

_B, _V, _S = 16, 1024, 2048
_L = 16
_W = 128
_G = _W // _L
_CH = 256
_NCH = _V // _CH

_info = plsc.get_sparse_core_info()
_NC, _NS = _info.num_cores, _info.num_subcores
_NW = _NC * _NS
_NSLAB = (_B * (_S // _W)) // _NW


def _enc(v):
    u = plsc.bitcast(v, jnp.uint32)
    s = u >> jnp.uint32(31)
    mask = s * jnp.uint32(0x7FFFFFFF) + jnp.uint32(0x80000000)
    return u ^ mask


def _dec(e):
    s = e >> jnp.uint32(31)
    mask = jnp.uint32(0xFFFFFFFF) - s * jnp.uint32(0x7FFFFFFF)
    return plsc.bitcast(e ^ mask, jnp.float32)


def _chunk(ref, b, t0, c):
    return ref.at[b, pl.ds(c * _CH, _CH), pl.ds(t0, _W)]


def _body(x_hbm, out_hbm, b0, b1, b2, is0, is1, is2, os0, os1, os2):
    wid = lax.axis_index("s") * _NC + lax.axis_index("c")
    bufs = (b0, b1, b2)
    isems = (is0, is1, is2)
    osems = (os0, os1, os2)

    def slab(j, carry):
        s = wid * _NSLAB + j
        b = s // (_S // _W)
        t0 = (s % (_S // _W)) * _W

        for c in range(3):
            pltpu.async_copy(_chunk(x_hbm, b, t0, c), bufs[c], isems[c])

        def p1(buf):
            def rows(i, mm):
                m1, m2 = mm
                base = i * 4
                for r in range(4):
                    vs = [buf[base + r, pl.ds(16 * g, 16)] for g in range(_G)]
                    mn = [jnp.minimum(m1[g], vs[g]) for g in range(_G)]
                    m1 = tuple(
                        jnp.maximum(m1[g], vs[g]) for g in range(_G)
                    )
                    m2 = tuple(jnp.maximum(m2[g], mn[g]) for g in range(_G))
                return m1, m2

            return rows

        ninf = tuple(jnp.full((_L,), -jnp.inf, jnp.float32) for _ in range(_G))
        mm = (ninf, ninf)
        for c in range(_NCH):
            pltpu.make_async_copy(
                _chunk(x_hbm, b, t0, c), bufs[c % 3], isems[c % 3]
            ).wait()
            if 3 <= c + 1 < _NCH:
                pltpu.async_copy(
                    _chunk(x_hbm, b, t0, c + 1), bufs[(c + 1) % 3],
                    isems[(c + 1) % 3],
                )
            mm = lax.fori_loop(0, _CH // 4, p1(bufs[c % 3]), mm)
        m1, m2 = mm

        em = [jnp.exp(m1[g]) for g in range(_G)]
        tie = [m2[g] == m1[g] for g in range(_G)]
        rare = [em[g] < TRUNC_R for g in range(_G)]
        n_slow = sum(
            plsc.all_reduce_population_count(tie[g] | rare[g])[0]
            for g in range(_G)
        )
        n_rare = sum(
            plsc.all_reduce_population_count(rare[g])[0] for g in range(_G)
        )

        def fast(_):
            def pf(buf):
                def rows(i, carry):
                    base = i * 4
                    for r in range(4):
                        for g in range(_G):
                            v = buf[base + r, pl.ds(16 * g, 16)]
                            buf[base + r, pl.ds(16 * g, 16)] = jnp.where(
                                v == m1[g], v, NEG_FILL
                            )
                    return carry

                return rows

            lax.fori_loop(0, _CH // 4, pf(b0), 0)
            pltpu.async_copy(b0, _chunk(out_hbm, b, t0, 3), os0)
            lax.fori_loop(0, _CH // 4, pf(b2), 0)
            pltpu.async_copy(b2, _chunk(out_hbm, b, t0, 2), os2)
            pltpu.make_async_copy(b0, _chunk(out_hbm, b, t0, 3), os0).wait()
            pltpu.async_copy(_chunk(x_hbm, b, t0, 0), b0, is0)
            lax.fori_loop(0, _CH // 4, pf(b1), 0)
            pltpu.async_copy(b1, _chunk(out_hbm, b, t0, 1), os1)
            pltpu.make_async_copy(_chunk(x_hbm, b, t0, 0), b0, is0).wait()
            lax.fori_loop(0, _CH // 4, pf(b0), 0)
            pltpu.async_copy(b0, _chunk(out_hbm, b, t0, 0), os0)
            pltpu.make_async_copy(b0, _chunk(out_hbm, b, t0, 0), os0).wait()
            pltpu.make_async_copy(b1, _chunk(out_hbm, b, t0, 1), os1).wait()
            pltpu.make_async_copy(b2, _chunk(out_hbm, b, t0, 2), os2).wait()
            return 0

        def slow(_):
            def mass_gt(thr):
                def rows(i, acc):
                    return tuple(
                        acc[g]
                        + jnp.where(
                            _enc(b0[i, pl.ds(16 * g, 16)]) > thr[g],
                            jnp.exp(b0[i, pl.ds(16 * g, 16)]),
                            0.0,
                        )
                        for g in range(_G)
                    )

                acc = tuple(jnp.zeros((_L,), jnp.float32) for _ in range(_G))
                for c in range(_NCH):
                    pltpu.sync_copy(_chunk(x_hbm, b, t0, c), b0)
                    acc = lax.fori_loop(0, _CH, rows, acc)
                return acc

            def common(_):
                return (
                    tuple(_enc(m1[g]) for g in range(_G)),
                    tuple(jnp.zeros((_L,), jnp.float32) for _ in range(_G)),
                )

            def search(_):
                def sb(_i, c):
                    lo, hi = c
                    mid = tuple(
                        lo[g] + ((hi[g] - lo[g]) >> jnp.uint32(1))
                        for g in range(_G)
                    )
                    pred = [mg < TRUNC_R for mg in mass_gt(mid)]
                    return (
                        tuple(
                            jnp.where(pred[g], lo[g], mid[g] + jnp.uint32(1))
                            for g in range(_G)
                        ),
                        tuple(
                            jnp.where(pred[g], mid[g], hi[g])
                            for g in range(_G)
                        ),
                    )

                lo0 = tuple(jnp.zeros((_L,), jnp.uint32) for _ in range(_G))
                hi0 = tuple(
                    jnp.full((_L,), 0xFFFFFFFF, jnp.uint32) for _ in range(_G)
                )
                _, thr = lax.fori_loop(0, 32, sb, (lo0, hi0))
                return thr, mass_gt(thr)

            T, A = lax.cond(n_rare > 0, search, common, None)
            e_t = [jnp.exp(_dec(T[g])) for g in range(_G)]

            def cnt_pass(buf):
                def rows(i, acc):
                    base = i * 2
                    for r in range(2):
                        acc = tuple(
                            acc[g]
                            + jnp.where(
                                _enc(buf[base + r, pl.ds(16 * g, 16)])
                                == T[g],
                                1.0,
                                0.0,
                            )
                            for g in range(_G)
                        )
                    return acc

                zero = tuple(
                    jnp.zeros((_L,), jnp.float32) for _ in range(_G)
                )
                return lax.fori_loop(0, _CH // 2, rows, zero)

            cnt1 = cnt_pass(b1)
            cnt2 = cnt_pass(b2)
            pltpu.sync_copy(_chunk(x_hbm, b, t0, 0), b0)
            cnt0 = cnt_pass(b0)
            s_c = [A]
            pref = cnt0
            for cn in (cnt1, cnt2):
                s_c.append(
                    tuple(
                        A[g] + e_t[g] * pref[g] for g in range(_G)
                    )
                )
                pref = tuple(pref[g] + cn[g] for g in range(_G))
            s_c.append(tuple(A[g] + e_t[g] * pref[g] for g in range(_G)))

            def p2(buf):
                def rows(i, ss):
                    out = []
                    for g in range(_G):
                        v = buf[i, pl.ds(16 * g, 16)]
                        ev = _enc(v)
                        eq = ev == T[g]
                        keep = (ev > T[g]) | (eq & (ss[g] < TRUNC_R))
                        buf[i, pl.ds(16 * g, 16)] = jnp.where(
                            keep, v, NEG_FILL
                        )
                        out.append(ss[g] + jnp.where(eq, e_t[g], 0.0))
                    return tuple(out)

                return rows

            lax.fori_loop(0, _CH, p2(b0), s_c[0])
            pltpu.sync_copy(b0, _chunk(out_hbm, b, t0, 0))
            lax.fori_loop(0, _CH, p2(b1), s_c[1])
            pltpu.sync_copy(b1, _chunk(out_hbm, b, t0, 1))
            lax.fori_loop(0, _CH, p2(b2), s_c[2])
            pltpu.sync_copy(b2, _chunk(out_hbm, b, t0, 2))
            pltpu.sync_copy(_chunk(x_hbm, b, t0, 3), b0)
            lax.fori_loop(0, _CH, p2(b0), s_c[3])
            pltpu.sync_copy(b0, _chunk(out_hbm, b, t0, 3))
            return 0

        lax.cond(n_slow > 0, slow, fast, None)
        return carry

    lax.fori_loop(0, _NSLAB, slab, 0)


_sc_call = functools.partial(
    pl.kernel,
    out_type=jax.ShapeDtypeStruct((_B, _V, _S), jnp.float32),
    mesh=plsc.VectorSubcoreMesh(core_axis_name="c", subcore_axis_name="s"),
    scratch_types=[
        pltpu.VMEM((_CH, _W), jnp.float32),
        pltpu.VMEM((_CH, _W), jnp.float32),
        pltpu.VMEM((_CH, _W), jnp.float32),
        pltpu.SemaphoreType.DMA,
        pltpu.SemaphoreType.DMA,
        pltpu.SemaphoreType.DMA,
        pltpu.SemaphoreType.DMA,
        pltpu.SemaphoreType.DMA,
        pltpu.SemaphoreType.DMA,
    ],
    compiler_params=pltpu.CompilerParams(needs_layout_passes=False),
)(_body)


@jax.jit
def kernel(logits):
    return _sc_call(logits)


# speedup vs baseline: 1.0193x; 1.0193x over previous
import functools

import jax
import jax.numpy as jnp
from jax import lax
from jax.experimental import pallas as pl
from jax.experimental.pallas import tpu as pltpu
from jax.experimental.pallas import tpu_sc as plsc

TRUNC_R = 0.85
NEG_FILL = -70.0

_B, _V, _S = 16, 1024, 2048
_L = 16
_W = 128
_G = _W // _L
_CH = 256
_NCH = _V // _CH

_info = plsc.get_sparse_core_info()
_NC, _NS = _info.num_cores, _info.num_subcores
_NW = _NC * _NS
_NSLAB = (_B * (_S // _W)) // _NW


def _enc(v):
    u = plsc.bitcast(v, jnp.uint32)
    s = u >> jnp.uint32(31)
    mask = s * jnp.uint32(0x7FFFFFFF) + jnp.uint32(0x80000000)
    return u ^ mask


def _dec(e):
    s = e >> jnp.uint32(31)
    mask = jnp.uint32(0xFFFFFFFF) - s * jnp.uint32(0x7FFFFFFF)
    return plsc.bitcast(e ^ mask, jnp.float32)


def _chunk(ref, b, t0, c):
    return ref.at[b, pl.ds(c * _CH, _CH), pl.ds(t0, _W)]


def _body(x_hbm, out_hbm, b0, b1, b2, is0, is1, is2, os0, os1, os2):
    wid = lax.axis_index("s") * _NC + lax.axis_index("c")
    bufs = (b0, b1, b2)
    isems = (is0, is1, is2)
    osems = (os0, os1, os2)

    def slab(j, carry):
        s = wid * _NSLAB + j
        b = s // (_S // _W)
        t0 = (s % (_S // _W)) * _W

        for c in range(3):
            pltpu.async_copy(_chunk(x_hbm, b, t0, c), bufs[c], isems[c])

        def p1(buf):
            def rows(i, mm):
                m1, m2 = mm
                base = i * 2
                for r in range(2):
                    vs = [buf[base + r, pl.ds(16 * g, 16)] for g in range(_G)]
                    mn = [jnp.minimum(m1[g], vs[g]) for g in range(_G)]
                    m1 = tuple(
                        jnp.maximum(m1[g], vs[g]) for g in range(_G)
                    )
                    m2 = tuple(jnp.maximum(m2[g], mn[g]) for g in range(_G))
                return m1, m2

            return rows

        ninf = tuple(jnp.full((_L,), -jnp.inf, jnp.float32) for _ in range(_G))
        mm = (ninf, ninf)
        for c in range(_NCH):
            pltpu.make_async_copy(
                _chunk(x_hbm, b, t0, c), bufs[c % 3], isems[c % 3]
            ).wait()
            if 3 <= c + 1 < _NCH:
                pltpu.async_copy(
                    _chunk(x_hbm, b, t0, c + 1), bufs[(c + 1) % 3],
                    isems[(c + 1) % 3],
                )
            mm = lax.fori_loop(0, _CH // 2, p1(bufs[c % 3]), mm)
        m1, m2 = mm

        em = [jnp.exp(m1[g]) for g in range(_G)]
        tie = [m2[g] == m1[g] for g in range(_G)]
        rare = [em[g] < TRUNC_R for g in range(_G)]
        n_slow = sum(
            plsc.all_reduce_population_count(tie[g] | rare[g])[0]
            for g in range(_G)
        )
        n_rare = sum(
            plsc.all_reduce_population_count(rare[g])[0] for g in range(_G)
        )

        def fast(_):
            def pf(buf):
                def rows(i, carry):
                    base = i * 2
                    for r in range(2):
                        for g in range(_G):
                            v = buf[base + r, pl.ds(16 * g, 16)]
                            buf[base + r, pl.ds(16 * g, 16)] = jnp.where(
                                v == m1[g], v, NEG_FILL
                            )
                    return carry

                return rows

            lax.fori_loop(0, _CH // 2, pf(b0), 0)
            pltpu.async_copy(b0, _chunk(out_hbm, b, t0, 3), os0)
            lax.fori_loop(0, _CH // 2, pf(b2), 0)
            pltpu.async_copy(b2, _chunk(out_hbm, b, t0, 2), os2)
            pltpu.make_async_copy(b0, _chunk(out_hbm, b, t0, 3), os0).wait()
            pltpu.async_copy(_chunk(x_hbm, b, t0, 0), b0, is0)
            lax.fori_loop(0, _CH // 2, pf(b1), 0)
            pltpu.async_copy(b1, _chunk(out_hbm, b, t0, 1), os1)
            pltpu.make_async_copy(_chunk(x_hbm, b, t0, 0), b0, is0).wait()
            lax.fori_loop(0, _CH // 2, pf(b0), 0)
            pltpu.async_copy(b0, _chunk(out_hbm, b, t0, 0), os0)
            pltpu.make_async_copy(b0, _chunk(out_hbm, b, t0, 0), os0).wait()
            pltpu.make_async_copy(b1, _chunk(out_hbm, b, t0, 1), os1).wait()
            pltpu.make_async_copy(b2, _chunk(out_hbm, b, t0, 2), os2).wait()
            return 0

        def slow(_):
            def mass_gt(thr):
                def rows(i, acc):
                    return tuple(
                        acc[g]
                        + jnp.where(
                            _enc(b0[i, pl.ds(16 * g, 16)]) > thr[g],
                            jnp.exp(b0[i, pl.ds(16 * g, 16)]),
                            0.0,
                        )
                        for g in range(_G)
                    )

                acc = tuple(jnp.zeros((_L,), jnp.float32) for _ in range(_G))
                for c in range(_NCH):
                    pltpu.sync_copy(_chunk(x_hbm, b, t0, c), b0)
                    acc = lax.fori_loop(0, _CH, rows, acc)
                return acc

            def common(_):
                return (
                    tuple(_enc(m1[g]) for g in range(_G)),
                    tuple(jnp.zeros((_L,), jnp.float32) for _ in range(_G)),
                )

            def search(_):
                def sb(_i, c):
                    lo, hi = c
                    mid = tuple(
                        lo[g] + ((hi[g] - lo[g]) >> jnp.uint32(1))
                        for g in range(_G)
                    )
                    pred = [mg < TRUNC_R for mg in mass_gt(mid)]
                    return (
                        tuple(
                            jnp.where(pred[g], lo[g], mid[g] + jnp.uint32(1))
                            for g in range(_G)
                        ),
                        tuple(
                            jnp.where(pred[g], mid[g], hi[g])
                            for g in range(_G)
                        ),
                    )

                lo0 = tuple(jnp.zeros((_L,), jnp.uint32) for _ in range(_G))
                hi0 = tuple(
                    jnp.full((_L,), 0xFFFFFFFF, jnp.uint32) for _ in range(_G)
                )
                _, thr = lax.fori_loop(0, 32, sb, (lo0, hi0))
                return thr, mass_gt(thr)

            T, A = lax.cond(n_rare > 0, search, common, None)
            e_t = [jnp.exp(_dec(T[g])) for g in range(_G)]

            def p2(i, ss):
                out = []
                for g in range(_G):
                    v = b0[i, pl.ds(16 * g, 16)]
                    ev = _enc(v)
                    eq = ev == T[g]
                    keep = (ev > T[g]) | (eq & (ss[g] < TRUNC_R))
                    b0[i, pl.ds(16 * g, 16)] = jnp.where(keep, v, NEG_FILL)
                    out.append(ss[g] + jnp.where(eq, e_t[g], 0.0))
                return tuple(out)

            ss = A
            for c in range(_NCH):
                pltpu.sync_copy(_chunk(x_hbm, b, t0, c), b0)
                ss = lax.fori_loop(0, _CH, p2, ss)
                pltpu.sync_copy(b0, _chunk(out_hbm, b, t0, c))
            return 0

        lax.cond(n_slow > 0, slow, fast, None)
        return carry

    lax.fori_loop(0, _NSLAB, slab, 0)


_sc_call = functools.partial(
    pl.kernel,
    out_type=jax.ShapeDtypeStruct((_B, _V, _S), jnp.float32),
    mesh=plsc.VectorSubcoreMesh(core_axis_name="c", subcore_axis_name="s"),
    scratch_types=[
        pltpu.VMEM((_CH, _W), jnp.float32),
        pltpu.VMEM((_CH, _W), jnp.float32),
        pltpu.VMEM((_CH, _W), jnp.float32),
        pltpu.SemaphoreType.DMA,
        pltpu.SemaphoreType.DMA,
        pltpu.SemaphoreType.DMA,
        pltpu.SemaphoreType.DMA,
        pltpu.SemaphoreType.DMA,
        pltpu.SemaphoreType.DMA,
    ],
    compiler_params=pltpu.CompilerParams(needs_layout_passes=False),
)(_body)


@jax.jit
def kernel(logits):
    return _sc_call(logits)


# speedup vs baseline: 1.0914x; 1.0708x over previous
import functools

import jax
import jax.numpy as jnp
from jax import lax
from jax.experimental import pallas as pl
from jax.experimental.pallas import tpu as pltpu
from jax.experimental.pallas import tpu_sc as plsc

TRUNC_R = 0.85
NEG_FILL = -70.0

_B, _V, _S = 16, 1024, 2048
_L = 16
_W = 128
_G = _W // _L
_CH = 256
_NCH = _V // _CH

_info = plsc.get_sparse_core_info()
_NC, _NS = _info.num_cores, _info.num_subcores
_NW = _NC * _NS
_NSLAB = (_B * (_S // _W)) // _NW


def _enc(v):
    u = plsc.bitcast(v, jnp.uint32)
    s = u >> jnp.uint32(31)
    mask = s * jnp.uint32(0x7FFFFFFF) + jnp.uint32(0x80000000)
    return u ^ mask


def _dec(e):
    s = e >> jnp.uint32(31)
    mask = jnp.uint32(0xFFFFFFFF) - s * jnp.uint32(0x7FFFFFFF)
    return plsc.bitcast(e ^ mask, jnp.float32)


def _chunk(ref, b, t0, c):
    return ref.at[b, pl.ds(c * _CH, _CH), pl.ds(t0, _W)]


def _body(x_hbm, out_hbm, b0, b1, b2, is0, is1, is2, os0, os1, os2):
    wid = lax.axis_index("s") * _NC + lax.axis_index("c")
    bufs = (b0, b1, b2)
    isems = (is0, is1, is2)
    osems = (os0, os1, os2)

    def slab(j, carry):
        s = wid * _NSLAB + j
        b = s // (_S // _W)
        t0 = (s % (_S // _W)) * _W

        prev = jnp.maximum(s - 1, 0)
        pb = prev // (_S // _W)
        pt0 = (prev % (_S // _W)) * _W
        for c in (2, 0, 1):
            @pl.when(j > 0)
            def _(c=c):
                pltpu.make_async_copy(
                    bufs[c], _chunk(out_hbm, pb, pt0, c), osems[c]
                ).wait()

            pltpu.async_copy(_chunk(x_hbm, b, t0, c), bufs[c], isems[c])

        def p1(buf):
            def rows(i, mm):
                m1, m2 = mm
                base = i * 2
                for r in range(2):
                    vs = [buf[base + r, pl.ds(16 * g, 16)] for g in range(_G)]
                    mn = [jnp.minimum(m1[g], vs[g]) for g in range(_G)]
                    m1 = tuple(
                        jnp.maximum(m1[g], vs[g]) for g in range(_G)
                    )
                    m2 = tuple(jnp.maximum(m2[g], mn[g]) for g in range(_G))
                return m1, m2

            return rows

        ninf = tuple(jnp.full((_L,), -jnp.inf, jnp.float32) for _ in range(_G))
        mm = (ninf, ninf)
        for c in (2, 0, 1, 3):
            pltpu.make_async_copy(
                _chunk(x_hbm, b, t0, c), bufs[c % 3], isems[c % 3]
            ).wait()
            mm = lax.fori_loop(0, _CH // 2, p1(bufs[c % 3]), mm)
            if c == 0:
                pltpu.async_copy(_chunk(x_hbm, b, t0, 3), b0, is0)
        m1, m2 = mm

        em = [jnp.exp(m1[g]) for g in range(_G)]
        tie = [m2[g] == m1[g] for g in range(_G)]
        rare = [em[g] < TRUNC_R for g in range(_G)]
        n_slow = sum(
            plsc.all_reduce_population_count(tie[g] | rare[g])[0]
            for g in range(_G)
        )
        n_rare = sum(
            plsc.all_reduce_population_count(rare[g])[0] for g in range(_G)
        )

        def fast(_):
            def pf(buf):
                def rows(i, carry):
                    base = i * 2
                    for r in range(2):
                        for g in range(_G):
                            v = buf[base + r, pl.ds(16 * g, 16)]
                            buf[base + r, pl.ds(16 * g, 16)] = jnp.where(
                                v == m1[g], v, NEG_FILL
                            )
                    return carry

                return rows

            lax.fori_loop(0, _CH // 2, pf(b0), 0)
            pltpu.async_copy(b0, _chunk(out_hbm, b, t0, 3), os0)
            lax.fori_loop(0, _CH // 2, pf(b2), 0)
            pltpu.async_copy(b2, _chunk(out_hbm, b, t0, 2), os2)
            pltpu.make_async_copy(b0, _chunk(out_hbm, b, t0, 3), os0).wait()
            pltpu.async_copy(_chunk(x_hbm, b, t0, 0), b0, is0)
            lax.fori_loop(0, _CH // 2, pf(b1), 0)
            pltpu.async_copy(b1, _chunk(out_hbm, b, t0, 1), os1)
            pltpu.make_async_copy(_chunk(x_hbm, b, t0, 0), b0, is0).wait()
            lax.fori_loop(0, _CH // 2, pf(b0), 0)
            pltpu.async_copy(b0, _chunk(out_hbm, b, t0, 0), os0)
            return 0

        def slow(_):
            def mass_gt(thr):
                def rows(i, acc):
                    return tuple(
                        acc[g]
                        + jnp.where(
                            _enc(b0[i, pl.ds(16 * g, 16)]) > thr[g],
                            jnp.exp(b0[i, pl.ds(16 * g, 16)]),
                            0.0,
                        )
                        for g in range(_G)
                    )

                acc = tuple(jnp.zeros((_L,), jnp.float32) for _ in range(_G))
                for c in range(_NCH):
                    pltpu.sync_copy(_chunk(x_hbm, b, t0, c), b0)
                    acc = lax.fori_loop(0, _CH, rows, acc)
                return acc

            def common(_):
                return (
                    tuple(_enc(m1[g]) for g in range(_G)),
                    tuple(jnp.zeros((_L,), jnp.float32) for _ in range(_G)),
                )

            def search(_):
                def sb(_i, c):
                    lo, hi = c
                    mid = tuple(
                        lo[g] + ((hi[g] - lo[g]) >> jnp.uint32(1))
                        for g in range(_G)
                    )
                    pred = [mg < TRUNC_R for mg in mass_gt(mid)]
                    return (
                        tuple(
                            jnp.where(pred[g], lo[g], mid[g] + jnp.uint32(1))
                            for g in range(_G)
                        ),
                        tuple(
                            jnp.where(pred[g], mid[g], hi[g])
                            for g in range(_G)
                        ),
                    )

                lo0 = tuple(jnp.zeros((_L,), jnp.uint32) for _ in range(_G))
                hi0 = tuple(
                    jnp.full((_L,), 0xFFFFFFFF, jnp.uint32) for _ in range(_G)
                )
                _, thr = lax.fori_loop(0, 32, sb, (lo0, hi0))
                return thr, mass_gt(thr)

            T, A = lax.cond(n_rare > 0, search, common, None)
            e_t = [jnp.exp(_dec(T[g])) for g in range(_G)]

            def cnt_pass(buf):
                def rows(i, acc):
                    return tuple(
                        acc[g]
                        + jnp.where(
                            _enc(buf[i, pl.ds(16 * g, 16)]) == T[g], 1.0, 0.0
                        )
                        for g in range(_G)
                    )

                zero = tuple(
                    jnp.zeros((_L,), jnp.float32) for _ in range(_G)
                )
                return lax.fori_loop(0, _CH, rows, zero)

            cnt1 = cnt_pass(b1)
            cnt2 = cnt_pass(b2)
            pltpu.sync_copy(_chunk(x_hbm, b, t0, 0), b0)
            cnt0 = cnt_pass(b0)
            pref01 = tuple(cnt0[g] + cnt1[g] for g in range(_G))
            s_c = (
                A,
                tuple(A[g] + e_t[g] * cnt0[g] for g in range(_G)),
                tuple(A[g] + e_t[g] * pref01[g] for g in range(_G)),
                tuple(
                    A[g] + e_t[g] * (pref01[g] + cnt2[g]) for g in range(_G)
                ),
            )

            def p2(buf):
                def rows(i, ss):
                    out = []
                    for g in range(_G):
                        v = buf[i, pl.ds(16 * g, 16)]
                        ev = _enc(v)
                        eq = ev == T[g]
                        keep = (ev > T[g]) | (eq & (ss[g] < TRUNC_R))
                        buf[i, pl.ds(16 * g, 16)] = jnp.where(
                            keep, v, NEG_FILL
                        )
                        out.append(ss[g] + jnp.where(eq, e_t[g], 0.0))
                    return tuple(out)

                return rows

            lax.fori_loop(0, _CH, p2(b0), s_c[0])
            pltpu.async_copy(b0, _chunk(out_hbm, b, t0, 0), os0)
            lax.fori_loop(0, _CH, p2(b1), s_c[1])
            pltpu.async_copy(b1, _chunk(out_hbm, b, t0, 1), os1)
            lax.fori_loop(0, _CH, p2(b2), s_c[2])
            pltpu.async_copy(b2, _chunk(out_hbm, b, t0, 2), os2)
            pltpu.make_async_copy(b0, _chunk(out_hbm, b, t0, 0), os0).wait()
            pltpu.sync_copy(_chunk(x_hbm, b, t0, 3), b0)
            lax.fori_loop(0, _CH, p2(b0), s_c[3])
            pltpu.async_copy(b0, _chunk(out_hbm, b, t0, 3), os0)
            return 0

        lax.cond(n_slow > 0, slow, fast, None)
        return carry

    lax.fori_loop(0, _NSLAB, slab, 0)
    last = wid * _NSLAB + _NSLAB - 1
    lb = last // (_S // _W)
    lt0 = (last % (_S // _W)) * _W
    for c in range(3):
        pltpu.make_async_copy(
            bufs[c], _chunk(out_hbm, lb, lt0, c), osems[c]
        ).wait()


_sc_call = functools.partial(
    pl.kernel,
    out_type=jax.ShapeDtypeStruct((_B, _V, _S), jnp.float32),
    mesh=plsc.VectorSubcoreMesh(core_axis_name="c", subcore_axis_name="s"),
    scratch_types=[
        pltpu.VMEM((_CH, _W), jnp.float32),
        pltpu.VMEM((_CH, _W), jnp.float32),
        pltpu.VMEM((_CH, _W), jnp.float32),
        pltpu.SemaphoreType.DMA,
        pltpu.SemaphoreType.DMA,
        pltpu.SemaphoreType.DMA,
        pltpu.SemaphoreType.DMA,
        pltpu.SemaphoreType.DMA,
        pltpu.SemaphoreType.DMA,
    ],
    compiler_params=pltpu.CompilerParams(needs_layout_passes=False),
)(_body)


@jax.jit
def kernel(logits):
    return _sc_call(logits)
